# baseline (speedup 1.0000x reference)
import functools
import math

import numpy as np
import jax
import jax.numpy as jnp
from jax import lax
from jax.experimental import pallas as pl
from jax.experimental.pallas import tpu as pltpu
from jax.experimental.pallas import tpu_sc as plsc

MUL = 32
DIM = 4 * MUL
SQRT2 = math.sqrt(2.0)
INV_SQRT3 = 1.0 / math.sqrt(3.0)
ALPHA = 1.0 / math.sqrt(2.0 * MUL)

TE = 256
CE = 128


def _build_consts():
    c = np.arange(32 * MUL)
    Rm = (c[None, :] // MUL == np.arange(MUL)[:, None]).astype(np.float32)
    Sm = (c[:, None] % MUL == np.arange(MUL)[None, :]).astype(np.float32)
    Qm = np.zeros((3 * MUL, 3 * MUL), np.float32)
    for i in range(MUL):
        for k in range(3):
            Qm[3 * i + k, k * MUL + i] = 1.0
    Pm = np.zeros((3 * MUL, 3 * MUL), np.float32)
    for j in range(MUL):
        for k in range(3):
            Pm[k * MUL + j, 3 * j + k] = 1.0
    return Rm, Sm, Qm, Pm

_R_NP, _S_NP, _Q_NP, _P_NP = _build_consts()


def _dense_body(r_ref, x_ref, sh_ref, w1_ref, w2_ref, R_ref, S_ref, Q_ref,
                P_ref, o_ref):
    f32 = jnp.float32
    hi = 'highest'
    r = r_ref[...]
    h = jnp.maximum(
        jnp.dot(r, w1_ref[...], precision=hi, preferred_element_type=f32),
        0.0) * SQRT2
    hb = h.astype(jnp.bfloat16)

    x = x_ref[...]
    sh = sh_ref[...]
    s = x[:, :MUL]
    xv = x[:, MUL:]
    y0 = sh[:, 0:1]

    Rm = R_ref[...]
    Sm = S_ref[...]
    v_all = jnp.dot(xv, Q_ref[...], precision=hi, preferred_element_type=f32)
    v0, v1, v2 = v_all[:, :MUL], v_all[:, MUL:2 * MUL], v_all[:, 2 * MUL:]
    dv = (v0 * sh[:, 1:2] + v1 * sh[:, 2:3] + v2 * sh[:, 3:4]) * INV_SQRT3

    def rep(z):
        return jnp.dot(z, Rm, precision=hi, preferred_element_type=f32)

    w2 = w2_ref[...]
    B = MUL * MUL

    def wblk(p):
        return jnp.dot(hb, w2[:, p * B:(p + 1) * B], preferred_element_type=f32)

    def contract(zrep, wp):
        return jnp.dot(zrep * wp, Sm, precision=hi, preferred_element_type=f32)

    srep = rep(s)
    q1 = contract(srep, wblk(0))
    q2 = contract(rep(dv), wblk(1))
    q3 = contract(srep, wblk(2))
    w4 = wblk(3)
    q40 = contract(rep(v0), w4)
    q41 = contract(rep(v1), w4)
    q42 = contract(rep(v2), w4)

    out_s = ALPHA * (q1 * y0 + q2)
    val = jnp.concatenate([
        ALPHA * (q3 * sh[:, 1:2] + q40 * y0),
        ALPHA * (q3 * sh[:, 2:3] + q41 * y0),
        ALPHA * (q3 * sh[:, 3:4] + q42 * y0),
    ], axis=1)
    out_vec = jnp.dot(val, P_ref[...], precision=hi, preferred_element_type=f32)
    o_ref[:, :MUL] = out_s
    o_ref[:, MUL:] = out_vec


def _dense_call(rp, x_e, sh, W1p, W2b, consts):
    E = rp.shape[0]
    HID = W1p.shape[1]
    Rm, Sm, Qm, Pm = consts
    grid = (E // TE,)
    return pl.pallas_call(
        _dense_body,
        grid=grid,
        in_specs=[
            pl.BlockSpec((TE, 16), lambda i: (i, 0)),
            pl.BlockSpec((TE, DIM), lambda i: (i, 0)),
            pl.BlockSpec((TE, 4), lambda i: (i, 0)),
            pl.BlockSpec((16, HID), lambda i: (0, 0)),
            pl.BlockSpec((HID, 4 * MUL * MUL), lambda i: (0, 0)),
            pl.BlockSpec((MUL, MUL * MUL), lambda i: (0, 0)),
            pl.BlockSpec((MUL * MUL, MUL), lambda i: (0, 0)),
            pl.BlockSpec((3 * MUL, 3 * MUL), lambda i: (0, 0)),
            pl.BlockSpec((3 * MUL, 3 * MUL), lambda i: (0, 0)),
        ],
        out_specs=pl.BlockSpec((TE, DIM), lambda i: (i, 0)),
        out_shape=jax.ShapeDtypeStruct((E, DIM), jnp.float32),
    )(rp, x_e, sh, W1p, W2b, Rm, Sm, Qm, Pm)


def _sc_gather(table, idx):
    E = idx.shape[0]
    n_chunks = E // CE
    mesh = plsc.VectorSubcoreMesh(core_axis_name="c", subcore_axis_name="s")
    NW = 32
    base_t, extra = divmod(n_chunks, NW)

    @functools.partial(
        pl.kernel,
        out_type=jax.ShapeDtypeStruct((E, DIM), jnp.float32),
        mesh=mesh,
        scratch_types=[
            pltpu.VMEM((CE,), jnp.int32),
            pltpu.VMEM((CE, DIM), jnp.float32),
            pltpu.SemaphoreType.DMA,
        ],
    )
    def gather_k(table_hbm, idx_hbm, out_hbm, idx_v, rows_v, sem):
        wid = lax.axis_index("s") * 2 + lax.axis_index("c")
        n_t = base_t + jnp.where(wid < extra, 1, 0)

        def body(t, carry):
            off = (wid + NW * t) * CE
            pltpu.sync_copy(idx_hbm.at[pl.ds(off, CE)], idx_v)
            pltpu.async_copy(table_hbm.at[idx_v], rows_v, sem).wait()
            pltpu.sync_copy(rows_v, out_hbm.at[pl.ds(off, CE)])
            return carry

        lax.fori_loop(0, n_t, body, 0)

    return gather_k(table, idx)


def _sc_scatter(tp, dst, zeros_nd):
    E = tp.shape[0]
    N = zeros_nd.shape[0]
    n_chunks = E // CE
    mesh = plsc.VectorSubcoreMesh(core_axis_name="c", subcore_axis_name="s")
    NW = 32
    NS = 16
    base_t, extra = divmod(n_chunks, NW)
    CR = 16
    base_u, extra_u = divmod(N // CR, NS)

    @functools.partial(
        pl.kernel,
        out_type=jax.ShapeDtypeStruct((2, N, DIM), jnp.float32),
        mesh=mesh,
        scratch_types=[
            pltpu.VMEM((CE,), jnp.int32),
            pltpu.VMEM((CE, DIM), jnp.float32),
            pltpu.VMEM_SHARED((N, DIM), jnp.float32),
            pltpu.SemaphoreType.DMA,
        ],
    )
    def scatter_k(tp_hbm, dst_hbm, zeros_hbm, out_hbm, idx_v, rows_v, acc_sh,
                  sem):
        cid = lax.axis_index("c")
        sid = lax.axis_index("s")
        wid = sid * 2 + cid

        @pl.when(sid == 0)
        def _():
            pltpu.sync_copy(zeros_hbm, acc_sh)

        plsc.subcore_barrier()

        n_t = base_t + jnp.where(wid < extra, 1, 0)

        def body(t, carry):
            off = (wid + NW * t) * CE
            pltpu.sync_copy(dst_hbm.at[pl.ds(off, CE)], idx_v)
            pltpu.sync_copy(tp_hbm.at[pl.ds(off, CE)], rows_v)
            pltpu.sync_copy(rows_v, acc_sh.at[idx_v], add=True)
            return carry

        lax.fori_loop(0, n_t, body, 0)
        plsc.subcore_barrier()

        n_u = base_u + jnp.where(sid < extra_u, 1, 0)

        def cbody(u, carry):
            roff = (sid + NS * u) * CR
            pltpu.sync_copy(acc_sh.at[pl.ds(roff, CR)],
                            out_hbm.at[cid, pl.ds(roff, CR)])
            return carry

        lax.fori_loop(0, n_u, cbody, 0)

    return scatter_k(tp, dst, zeros_nd)


def _combine_body(p_ref, s_ref, o_ref):
    o_ref[...] = (p_ref[0] + p_ref[1]) / s_ref[0, 0]


def _combine(partials, sq):
    N = partials.shape[1]
    BN = 1000
    return pl.pallas_call(
        _combine_body,
        grid=(N // BN,),
        in_specs=[
            pl.BlockSpec((2, BN, DIM), lambda i: (0, i, 0)),
            pl.BlockSpec(memory_space=pltpu.SMEM),
        ],
        out_specs=pl.BlockSpec((BN, DIM), lambda i: (i, 0)),
        out_shape=jax.ShapeDtypeStruct((N, DIM), jnp.float32),
    )(partials, sq)


def kernel(node_features, edge_sh_features, edge_radial_features, edge_src,
           edge_dst, n_nodes, W1, W2):
    N = node_features.shape[0]
    RAD = edge_radial_features.shape[1]
    HID = W1.shape[1]

    rp = jnp.pad(edge_radial_features, ((0, 0), (0, 16 - RAD)))
    W1p = jnp.pad(W1 * (1.0 / math.sqrt(float(RAD))), ((0, 16 - RAD), (0, 0)))
    W2b = (W2 * (1.0 / math.sqrt(float(HID)))).astype(jnp.bfloat16)
    consts = (jnp.asarray(_R_NP), jnp.asarray(_S_NP), jnp.asarray(_Q_NP),
              jnp.asarray(_P_NP))

    x_e = _sc_gather(node_features, edge_src)
    tp = _dense_call(rp, x_e, edge_sh_features, W1p, W2b, consts)
    partials = _sc_scatter(tp, edge_dst, jnp.zeros((N, DIM), jnp.float32))
    sq = jnp.sqrt((jnp.asarray(n_nodes) - 1).astype(jnp.float32)).reshape(1, 1)
    return _combine(partials, sq)


# speedup vs baseline: 2.3861x; 2.3861x over previous
import functools
import math

import numpy as np
import jax
import jax.numpy as jnp
from jax import lax
from jax.experimental import pallas as pl
from jax.experimental.pallas import tpu as pltpu
from jax.experimental.pallas import tpu_sc as plsc

MUL = 32
DIM = 4 * MUL
SQRT2 = math.sqrt(2.0)
INV_SQRT3 = 1.0 / math.sqrt(3.0)
ALPHA = 1.0 / math.sqrt(2.0 * MUL)

TE = 256
CE = 128


def _build_consts():
    c = np.arange(32 * MUL)
    Rm = (c[None, :] // MUL == np.arange(MUL)[:, None]).astype(np.float32)
    Sm = (c[:, None] % MUL == np.arange(MUL)[None, :]).astype(np.float32)
    Qm = np.zeros((3 * MUL, 3 * MUL), np.float32)
    for i in range(MUL):
        for k in range(3):
            Qm[3 * i + k, k * MUL + i] = 1.0
    Pm = np.zeros((3 * MUL, 3 * MUL), np.float32)
    for j in range(MUL):
        for k in range(3):
            Pm[k * MUL + j, 3 * j + k] = 1.0
    return Rm, Sm, Qm, Pm

_R_NP, _S_NP, _Q_NP, _P_NP = _build_consts()


def _dense_body(r_ref, x_ref, sh_ref, w1_ref, w2_ref, R_ref, S_ref, Q_ref,
                P_ref, o_ref):
    f32 = jnp.float32
    hi = None
    r = r_ref[...]
    h = jnp.maximum(
        jnp.dot(r, w1_ref[...], precision=hi, preferred_element_type=f32),
        0.0) * SQRT2
    hb = h.astype(jnp.bfloat16)

    x = x_ref[...]
    sh = sh_ref[...]
    s = x[:, :MUL]
    xv = x[:, MUL:]
    y0 = sh[:, 0:1]

    Rm = R_ref[...]
    Sm = S_ref[...]
    v_all = jnp.dot(xv, Q_ref[...], precision=hi, preferred_element_type=f32)
    v0, v1, v2 = v_all[:, :MUL], v_all[:, MUL:2 * MUL], v_all[:, 2 * MUL:]
    dv = (v0 * sh[:, 1:2] + v1 * sh[:, 2:3] + v2 * sh[:, 3:4]) * INV_SQRT3

    def rep(z):
        return jnp.dot(z, Rm, precision=hi, preferred_element_type=f32)

    w2 = w2_ref[...]
    B = MUL * MUL

    def wblk(p):
        return jnp.dot(hb, w2[:, p * B:(p + 1) * B], preferred_element_type=f32)

    def contract(zrep, wp):
        return jnp.dot(zrep * wp, Sm, precision=hi, preferred_element_type=f32)

    srep = rep(s)
    q1 = contract(srep, wblk(0))
    q2 = contract(rep(dv), wblk(1))
    q3 = contract(srep, wblk(2))
    w4 = wblk(3)
    q40 = contract(rep(v0), w4)
    q41 = contract(rep(v1), w4)
    q42 = contract(rep(v2), w4)

    out_s = ALPHA * (q1 * y0 + q2)
    val = jnp.concatenate([
        ALPHA * (q3 * sh[:, 1:2] + q40 * y0),
        ALPHA * (q3 * sh[:, 2:3] + q41 * y0),
        ALPHA * (q3 * sh[:, 3:4] + q42 * y0),
    ], axis=1)
    out_vec = jnp.dot(val, P_ref[...], precision=hi, preferred_element_type=f32)
    o_ref[:, :MUL] = out_s
    o_ref[:, MUL:] = out_vec


def _dense_call(rp, x_e, sh, W1p, W2b, consts):
    E = rp.shape[0]
    HID = W1p.shape[1]
    Rm, Sm, Qm, Pm = consts
    grid = (E // TE,)
    return pl.pallas_call(
        _dense_body,
        grid=grid,
        in_specs=[
            pl.BlockSpec((TE, 16), lambda i: (i, 0)),
            pl.BlockSpec((TE, DIM), lambda i: (i, 0)),
            pl.BlockSpec((TE, 4), lambda i: (i, 0)),
            pl.BlockSpec((16, HID), lambda i: (0, 0)),
            pl.BlockSpec((HID, 4 * MUL * MUL), lambda i: (0, 0)),
            pl.BlockSpec((MUL, MUL * MUL), lambda i: (0, 0)),
            pl.BlockSpec((MUL * MUL, MUL), lambda i: (0, 0)),
            pl.BlockSpec((3 * MUL, 3 * MUL), lambda i: (0, 0)),
            pl.BlockSpec((3 * MUL, 3 * MUL), lambda i: (0, 0)),
        ],
        out_specs=pl.BlockSpec((TE, DIM), lambda i: (i, 0)),
        out_shape=jax.ShapeDtypeStruct((E, DIM), jnp.float32),
    )(rp, x_e, sh, W1p, W2b, Rm, Sm, Qm, Pm)


def _sc_gather(table, idx):
    E = idx.shape[0]
    n_chunks = E // CE
    mesh = plsc.VectorSubcoreMesh(core_axis_name="c", subcore_axis_name="s")
    NW = 32
    base_t, extra = divmod(n_chunks, NW)

    @functools.partial(
        pl.kernel,
        out_type=jax.ShapeDtypeStruct((E, DIM), jnp.float32),
        mesh=mesh,
        scratch_types=[
            pltpu.VMEM((CE,), jnp.int32),
            pltpu.VMEM((CE, DIM), jnp.float32),
            pltpu.SemaphoreType.DMA,
        ],
    )
    def gather_k(table_hbm, idx_hbm, out_hbm, idx_v, rows_v, sem):
        wid = lax.axis_index("s") * 2 + lax.axis_index("c")
        n_t = base_t + jnp.where(wid < extra, 1, 0)

        def body(t, carry):
            off = (wid + NW * t) * CE
            pltpu.sync_copy(idx_hbm.at[pl.ds(off, CE)], idx_v)
            pltpu.async_copy(table_hbm.at[idx_v], rows_v, sem).wait()
            pltpu.sync_copy(rows_v, out_hbm.at[pl.ds(off, CE)])
            return carry

        lax.fori_loop(0, n_t, body, 0)

    return gather_k(table, idx)


def _sc_scatter(tp, dst, zeros_nd):
    E = tp.shape[0]
    N = zeros_nd.shape[0]
    n_chunks = E // CE
    mesh = plsc.VectorSubcoreMesh(core_axis_name="c", subcore_axis_name="s")
    NW = 32
    NS = 16
    base_t, extra = divmod(n_chunks, NW)
    CR = 16
    base_u, extra_u = divmod(N // CR, NS)

    @functools.partial(
        pl.kernel,
        out_type=jax.ShapeDtypeStruct((2, N, DIM), jnp.float32),
        mesh=mesh,
        scratch_types=[
            pltpu.VMEM((CE,), jnp.int32),
            pltpu.VMEM((CE, DIM), jnp.float32),
            pltpu.VMEM_SHARED((N, DIM), jnp.float32),
            pltpu.SemaphoreType.DMA,
        ],
    )
    def scatter_k(tp_hbm, dst_hbm, zeros_hbm, out_hbm, idx_v, rows_v, acc_sh,
                  sem):
        cid = lax.axis_index("c")
        sid = lax.axis_index("s")
        wid = sid * 2 + cid

        @pl.when(sid == 0)
        def _():
            pltpu.sync_copy(zeros_hbm, acc_sh)

        plsc.subcore_barrier()

        n_t = base_t + jnp.where(wid < extra, 1, 0)

        def body(t, carry):
            off = (wid + NW * t) * CE
            pltpu.sync_copy(dst_hbm.at[pl.ds(off, CE)], idx_v)
            pltpu.sync_copy(tp_hbm.at[pl.ds(off, CE)], rows_v)
            pltpu.sync_copy(rows_v, acc_sh.at[idx_v], add=True)
            return carry

        lax.fori_loop(0, n_t, body, 0)
        plsc.subcore_barrier()

        n_u = base_u + jnp.where(sid < extra_u, 1, 0)

        def cbody(u, carry):
            roff = (sid + NS * u) * CR
            pltpu.sync_copy(acc_sh.at[pl.ds(roff, CR)],
                            out_hbm.at[cid, pl.ds(roff, CR)])
            return carry

        lax.fori_loop(0, n_u, cbody, 0)

    return scatter_k(tp, dst, zeros_nd)


def _combine_body(p_ref, s_ref, o_ref):
    o_ref[...] = (p_ref[0] + p_ref[1]) / s_ref[0, 0]


def _combine(partials, sq):
    N = partials.shape[1]
    BN = 1000
    return pl.pallas_call(
        _combine_body,
        grid=(N // BN,),
        in_specs=[
            pl.BlockSpec((2, BN, DIM), lambda i: (0, i, 0)),
            pl.BlockSpec(memory_space=pltpu.SMEM),
        ],
        out_specs=pl.BlockSpec((BN, DIM), lambda i: (i, 0)),
        out_shape=jax.ShapeDtypeStruct((N, DIM), jnp.float32),
    )(partials, sq)


def kernel(node_features, edge_sh_features, edge_radial_features, edge_src,
           edge_dst, n_nodes, W1, W2):
    N = node_features.shape[0]
    RAD = edge_radial_features.shape[1]
    HID = W1.shape[1]

    rp = jnp.pad(edge_radial_features, ((0, 0), (0, 16 - RAD)))
    W1p = jnp.pad(W1 * (1.0 / math.sqrt(float(RAD))), ((0, 16 - RAD), (0, 0)))
    W2b = (W2 * (1.0 / math.sqrt(float(HID)))).astype(jnp.bfloat16)
    consts = (jnp.asarray(_R_NP), jnp.asarray(_S_NP), jnp.asarray(_Q_NP),
              jnp.asarray(_P_NP))

    x_e = _sc_gather(node_features, edge_src)
    tp = _dense_call(rp, x_e, edge_sh_features, W1p, W2b, consts)
    partials = _sc_scatter(tp, edge_dst, jnp.zeros((N, DIM), jnp.float32))
    sq = jnp.sqrt((jnp.asarray(n_nodes) - 1).astype(jnp.float32)).reshape(1, 1)
    return _combine(partials, sq)


# speedup vs baseline: 2.4826x; 1.0404x over previous
import functools
import math

import numpy as np
import jax
import jax.numpy as jnp
from jax import lax
from jax.experimental import pallas as pl
from jax.experimental.pallas import tpu as pltpu
from jax.experimental.pallas import tpu_sc as plsc

MUL = 32
DIM = 4 * MUL
SQRT2 = math.sqrt(2.0)
INV_SQRT3 = 1.0 / math.sqrt(3.0)
ALPHA = 1.0 / math.sqrt(2.0 * MUL)

TE = 512
CE = 128


def _build_consts():
    c = np.arange(32 * MUL)
    Rm = (c[None, :] // MUL == np.arange(MUL)[:, None]).astype(np.float32)
    Sm = (c[:, None] % MUL == np.arange(MUL)[None, :]).astype(np.float32)
    Qm = np.zeros((3 * MUL, 3 * MUL), np.float32)
    for i in range(MUL):
        for k in range(3):
            Qm[3 * i + k, k * MUL + i] = 1.0
    Pm = np.zeros((3 * MUL, 3 * MUL), np.float32)
    for j in range(MUL):
        for k in range(3):
            Pm[k * MUL + j, 3 * j + k] = 1.0
    return Rm, Sm, Qm, Pm

_R_NP, _S_NP, _Q_NP, _P_NP = _build_consts()


def _dense_body(r_ref, x_ref, sh_ref, w1_ref, w2_ref, R_ref, S_ref, Q_ref,
                P_ref, o_ref):
    f32 = jnp.float32
    hi = None
    r = r_ref[...]
    h = jnp.maximum(
        jnp.dot(r, w1_ref[...], precision=hi, preferred_element_type=f32),
        0.0) * SQRT2
    hb = h.astype(jnp.bfloat16)

    x = x_ref[...]
    sh = sh_ref[...]
    s = x[:, :MUL]
    xv = x[:, MUL:]
    y0 = sh[:, 0:1]

    Rm = R_ref[...]
    Sm = S_ref[...]
    v_all = jnp.dot(xv, Q_ref[...], precision=hi, preferred_element_type=f32)
    v0, v1, v2 = v_all[:, :MUL], v_all[:, MUL:2 * MUL], v_all[:, 2 * MUL:]
    dv = (v0 * sh[:, 1:2] + v1 * sh[:, 2:3] + v2 * sh[:, 3:4]) * INV_SQRT3

    def rep(z):
        return jnp.dot(z, Rm, precision=hi, preferred_element_type=f32)

    w2 = w2_ref[...]
    B = MUL * MUL

    def wblk(p):
        return jnp.dot(hb, w2[:, p * B:(p + 1) * B], preferred_element_type=f32)

    def contract(zrep, wp):
        return jnp.dot(zrep * wp, Sm, precision=hi, preferred_element_type=f32)

    srep = rep(s)
    q1 = contract(srep, wblk(0))
    q2 = contract(rep(dv), wblk(1))
    q3 = contract(srep, wblk(2))
    w4 = wblk(3)
    q40 = contract(rep(v0), w4)
    q41 = contract(rep(v1), w4)
    q42 = contract(rep(v2), w4)

    out_s = ALPHA * (q1 * y0 + q2)
    val = jnp.concatenate([
        ALPHA * (q3 * sh[:, 1:2] + q40 * y0),
        ALPHA * (q3 * sh[:, 2:3] + q41 * y0),
        ALPHA * (q3 * sh[:, 3:4] + q42 * y0),
    ], axis=1)
    out_vec = jnp.dot(val, P_ref[...], precision=hi, preferred_element_type=f32)
    o_ref[:, :MUL] = out_s
    o_ref[:, MUL:] = out_vec


def _dense_call(rp, x_e, sh, W1p, W2b, consts):
    E = rp.shape[0]
    HID = W1p.shape[1]
    Rm, Sm, Qm, Pm = consts
    grid = (E // TE,)
    return pl.pallas_call(
        _dense_body,
        grid=grid,
        in_specs=[
            pl.BlockSpec((TE, 16), lambda i: (i, 0)),
            pl.BlockSpec((TE, DIM), lambda i: (i, 0)),
            pl.BlockSpec((TE, 4), lambda i: (i, 0)),
            pl.BlockSpec((16, HID), lambda i: (0, 0)),
            pl.BlockSpec((HID, 4 * MUL * MUL), lambda i: (0, 0)),
            pl.BlockSpec((MUL, MUL * MUL), lambda i: (0, 0)),
            pl.BlockSpec((MUL * MUL, MUL), lambda i: (0, 0)),
            pl.BlockSpec((3 * MUL, 3 * MUL), lambda i: (0, 0)),
            pl.BlockSpec((3 * MUL, 3 * MUL), lambda i: (0, 0)),
        ],
        out_specs=pl.BlockSpec((TE, DIM), lambda i: (i, 0)),
        out_shape=jax.ShapeDtypeStruct((E, DIM), jnp.float32),
    )(rp, x_e, sh, W1p, W2b, Rm, Sm, Qm, Pm)


def _sc_gather(table, idx):
    E = idx.shape[0]
    n_chunks = E // CE
    mesh = plsc.VectorSubcoreMesh(core_axis_name="c", subcore_axis_name="s")
    NW = 32
    base_t, extra = divmod(n_chunks, NW)

    @functools.partial(
        pl.kernel,
        out_type=jax.ShapeDtypeStruct((E, DIM), jnp.float32),
        mesh=mesh,
        scratch_types=[
            pltpu.VMEM((CE,), jnp.int32),
            pltpu.VMEM((CE, DIM), jnp.float32),
            pltpu.SemaphoreType.DMA,
        ],
    )
    def gather_k(table_hbm, idx_hbm, out_hbm, idx_v, rows_v, sem):
        wid = lax.axis_index("s") * 2 + lax.axis_index("c")
        n_t = base_t + jnp.where(wid < extra, 1, 0)

        def body(t, carry):
            off = (wid + NW * t) * CE
            pltpu.sync_copy(idx_hbm.at[pl.ds(off, CE)], idx_v)
            pltpu.async_copy(table_hbm.at[idx_v], rows_v, sem).wait()
            pltpu.sync_copy(rows_v, out_hbm.at[pl.ds(off, CE)])
            return carry

        lax.fori_loop(0, n_t, body, 0)

    return gather_k(table, idx)


def _sc_scatter(tp, dst, zeros_nd):
    E = tp.shape[0]
    N = zeros_nd.shape[0]
    n_chunks = E // CE
    mesh = plsc.VectorSubcoreMesh(core_axis_name="c", subcore_axis_name="s")
    NW = 32
    NS = 16
    base_t, extra = divmod(n_chunks, NW)
    CR = 16
    base_u, extra_u = divmod(N // CR, NS)

    @functools.partial(
        pl.kernel,
        out_type=jax.ShapeDtypeStruct((2, N, DIM), jnp.float32),
        mesh=mesh,
        scratch_types=[
            pltpu.VMEM((CE,), jnp.int32),
            pltpu.VMEM((CE, DIM), jnp.float32),
            pltpu.VMEM_SHARED((N, DIM), jnp.float32),
            pltpu.SemaphoreType.DMA,
        ],
    )
    def scatter_k(tp_hbm, dst_hbm, zeros_hbm, out_hbm, idx_v, rows_v, acc_sh,
                  sem):
        cid = lax.axis_index("c")
        sid = lax.axis_index("s")
        wid = sid * 2 + cid

        @pl.when(sid == 0)
        def _():
            pltpu.sync_copy(zeros_hbm, acc_sh)

        plsc.subcore_barrier()

        n_t = base_t + jnp.where(wid < extra, 1, 0)

        def body(t, carry):
            off = (wid + NW * t) * CE
            pltpu.sync_copy(dst_hbm.at[pl.ds(off, CE)], idx_v)
            pltpu.sync_copy(tp_hbm.at[pl.ds(off, CE)], rows_v)
            pltpu.sync_copy(rows_v, acc_sh.at[idx_v], add=True)
            return carry

        lax.fori_loop(0, n_t, body, 0)
        plsc.subcore_barrier()

        n_u = base_u + jnp.where(sid < extra_u, 1, 0)

        def cbody(u, carry):
            roff = (sid + NS * u) * CR
            pltpu.sync_copy(acc_sh.at[pl.ds(roff, CR)],
                            out_hbm.at[cid, pl.ds(roff, CR)])
            return carry

        lax.fori_loop(0, n_u, cbody, 0)

    return scatter_k(tp, dst, zeros_nd)


def _combine_body(p_ref, s_ref, o_ref):
    o_ref[...] = (p_ref[0] + p_ref[1]) / s_ref[0, 0]


def _combine(partials, sq):
    N = partials.shape[1]
    BN = 1000
    return pl.pallas_call(
        _combine_body,
        grid=(N // BN,),
        in_specs=[
            pl.BlockSpec((2, BN, DIM), lambda i: (0, i, 0)),
            pl.BlockSpec(memory_space=pltpu.SMEM),
        ],
        out_specs=pl.BlockSpec((BN, DIM), lambda i: (i, 0)),
        out_shape=jax.ShapeDtypeStruct((N, DIM), jnp.float32),
    )(partials, sq)


def kernel(node_features, edge_sh_features, edge_radial_features, edge_src,
           edge_dst, n_nodes, W1, W2):
    N = node_features.shape[0]
    RAD = edge_radial_features.shape[1]
    HID = W1.shape[1]

    rp = jnp.pad(edge_radial_features, ((0, 0), (0, 16 - RAD)))
    W1p = jnp.pad(W1 * (1.0 / math.sqrt(float(RAD))), ((0, 16 - RAD), (0, 0)))
    W2b = (W2 * (1.0 / math.sqrt(float(HID)))).astype(jnp.bfloat16)
    consts = (jnp.asarray(_R_NP), jnp.asarray(_S_NP), jnp.asarray(_Q_NP),
              jnp.asarray(_P_NP))

    x_e = _sc_gather(node_features, edge_src)
    tp = _dense_call(rp, x_e, edge_sh_features, W1p, W2b, consts)
    partials = _sc_scatter(tp, edge_dst, jnp.zeros((N, DIM), jnp.float32))
    sq = jnp.sqrt((jnp.asarray(n_nodes) - 1).astype(jnp.float32)).reshape(1, 1)
    return _combine(partials, sq)


# speedup vs baseline: 2.7208x; 1.0959x over previous
import functools
import math

import numpy as np
import jax
import jax.numpy as jnp
from jax import lax
from jax.experimental import pallas as pl
from jax.experimental.pallas import tpu as pltpu
from jax.experimental.pallas import tpu_sc as plsc

MUL = 32
DIM = 4 * MUL
SQRT2 = math.sqrt(2.0)
INV_SQRT3 = 1.0 / math.sqrt(3.0)
ALPHA = 1.0 / math.sqrt(2.0 * MUL)

TE = 512
CE = 128


def _build_consts():
    c = np.arange(32 * MUL)
    Sm = (c[:, None] // MUL == np.arange(MUL)[None, :]).astype(np.float32)
    Qm = np.zeros((3 * MUL, 3 * MUL), np.float32)
    for i in range(MUL):
        for k in range(3):
            Qm[3 * i + k, k * MUL + i] = 1.0
    Pm = np.zeros((3 * MUL, 3 * MUL), np.float32)
    for j in range(MUL):
        for k in range(3):
            Pm[k * MUL + j, 3 * j + k] = 1.0
    perm = np.concatenate(
        [p * MUL * MUL + (c % MUL) * MUL + c // MUL for p in range(4)])
    return Sm, Qm, Pm, perm

_S_NP, _Q_NP, _P_NP, _PERM_NP = _build_consts()


def _dense_body(r_ref, x_ref, sh_ref, w1_ref, w2_ref, S_ref, Q_ref,
                P_ref, o_ref):
    f32 = jnp.float32
    bf16 = jnp.bfloat16
    rb = r_ref[...].astype(bf16)
    hb = jnp.maximum(
        jnp.dot(rb, w1_ref[...], preferred_element_type=f32),
        0.0).astype(bf16)

    x = x_ref[...]
    sh = sh_ref[...]
    s = x[:, :MUL]
    xv = x[:, MUL:].astype(bf16)
    y0 = sh[:, 0:1]

    Sm = S_ref[...]
    v_all = jnp.dot(xv, Q_ref[...], preferred_element_type=f32)
    v0, v1, v2 = v_all[:, :MUL], v_all[:, MUL:2 * MUL], v_all[:, 2 * MUL:]
    dv = (v0 * sh[:, 1:2] + v1 * sh[:, 2:3] + v2 * sh[:, 3:4]) * INV_SQRT3

    def rep(z):
        return pltpu.repeat(z, MUL, axis=1)

    w2 = w2_ref[...]
    B = MUL * MUL

    def wblk(p):
        return jnp.dot(hb, w2[:, p * B:(p + 1) * B], preferred_element_type=f32)

    def contract(zrep, wp):
        return jnp.dot((zrep * wp).astype(bf16), Sm, preferred_element_type=f32)

    srep = rep(s)
    q1 = contract(srep, wblk(0))
    q2 = contract(rep(dv), wblk(1))
    q3 = contract(srep, wblk(2))
    w4 = wblk(3)
    q40 = contract(rep(v0), w4)
    q41 = contract(rep(v1), w4)
    q42 = contract(rep(v2), w4)

    out_s = ALPHA * (q1 * y0 + q2)
    val = jnp.concatenate([
        ALPHA * (q3 * sh[:, 1:2] + q40 * y0),
        ALPHA * (q3 * sh[:, 2:3] + q41 * y0),
        ALPHA * (q3 * sh[:, 3:4] + q42 * y0),
    ], axis=1).astype(bf16)
    out_vec = jnp.dot(val, P_ref[...], preferred_element_type=f32)
    o_ref[:, :MUL] = out_s
    o_ref[:, MUL:] = out_vec


def _dense_call(rp, x_e, sh, W1p, W2b, consts):
    E = rp.shape[0]
    HID = W1p.shape[1]
    Sm, Qm, Pm = consts
    grid = (E // TE,)
    return pl.pallas_call(
        _dense_body,
        grid=grid,
        in_specs=[
            pl.BlockSpec((TE, 16), lambda i: (i, 0)),
            pl.BlockSpec((TE, DIM), lambda i: (i, 0)),
            pl.BlockSpec((TE, 4), lambda i: (i, 0)),
            pl.BlockSpec((16, HID), lambda i: (0, 0)),
            pl.BlockSpec((HID, 4 * MUL * MUL), lambda i: (0, 0)),
            pl.BlockSpec((MUL * MUL, MUL), lambda i: (0, 0)),
            pl.BlockSpec((3 * MUL, 3 * MUL), lambda i: (0, 0)),
            pl.BlockSpec((3 * MUL, 3 * MUL), lambda i: (0, 0)),
        ],
        out_specs=pl.BlockSpec((TE, DIM), lambda i: (i, 0)),
        out_shape=jax.ShapeDtypeStruct((E, DIM), jnp.float32),
    )(rp, x_e, sh, W1p, W2b, Sm, Qm, Pm)


def _sc_gather(table, idx):
    E = idx.shape[0]
    n_chunks = E // CE
    mesh = plsc.VectorSubcoreMesh(core_axis_name="c", subcore_axis_name="s")
    NW = 32
    base_t, extra = divmod(n_chunks, NW)

    @functools.partial(
        pl.kernel,
        out_type=jax.ShapeDtypeStruct((E, DIM), jnp.float32),
        mesh=mesh,
        scratch_types=[
            pltpu.VMEM((CE,), jnp.int32),
            pltpu.VMEM((CE, DIM), jnp.float32),
            pltpu.SemaphoreType.DMA,
        ],
    )
    def gather_k(table_hbm, idx_hbm, out_hbm, idx_v, rows_v, sem):
        wid = lax.axis_index("s") * 2 + lax.axis_index("c")
        n_t = base_t + jnp.where(wid < extra, 1, 0)

        def body(t, carry):
            off = (wid + NW * t) * CE
            pltpu.sync_copy(idx_hbm.at[pl.ds(off, CE)], idx_v)
            pltpu.async_copy(table_hbm.at[idx_v], rows_v, sem).wait()
            pltpu.sync_copy(rows_v, out_hbm.at[pl.ds(off, CE)])
            return carry

        lax.fori_loop(0, n_t, body, 0)

    return gather_k(table, idx)


def _sc_scatter(tp, dst, zeros_nd):
    E = tp.shape[0]
    N = zeros_nd.shape[0]
    n_chunks = E // CE
    mesh = plsc.VectorSubcoreMesh(core_axis_name="c", subcore_axis_name="s")
    NW = 32
    NS = 16
    base_t, extra = divmod(n_chunks, NW)
    CR = 16
    base_u, extra_u = divmod(N // CR, NS)

    @functools.partial(
        pl.kernel,
        out_type=jax.ShapeDtypeStruct((2, N, DIM), jnp.float32),
        mesh=mesh,
        scratch_types=[
            pltpu.VMEM((CE,), jnp.int32),
            pltpu.VMEM((CE, DIM), jnp.float32),
            pltpu.VMEM_SHARED((N, DIM), jnp.float32),
            pltpu.SemaphoreType.DMA,
        ],
    )
    def scatter_k(tp_hbm, dst_hbm, zeros_hbm, out_hbm, idx_v, rows_v, acc_sh,
                  sem):
        cid = lax.axis_index("c")
        sid = lax.axis_index("s")
        wid = sid * 2 + cid

        @pl.when(sid == 0)
        def _():
            pltpu.sync_copy(zeros_hbm, acc_sh)

        plsc.subcore_barrier()

        n_t = base_t + jnp.where(wid < extra, 1, 0)

        def body(t, carry):
            off = (wid + NW * t) * CE
            pltpu.sync_copy(dst_hbm.at[pl.ds(off, CE)], idx_v)
            pltpu.sync_copy(tp_hbm.at[pl.ds(off, CE)], rows_v)
            pltpu.sync_copy(rows_v, acc_sh.at[idx_v], add=True)
            return carry

        lax.fori_loop(0, n_t, body, 0)
        plsc.subcore_barrier()

        n_u = base_u + jnp.where(sid < extra_u, 1, 0)

        def cbody(u, carry):
            roff = (sid + NS * u) * CR
            pltpu.sync_copy(acc_sh.at[pl.ds(roff, CR)],
                            out_hbm.at[cid, pl.ds(roff, CR)])
            return carry

        lax.fori_loop(0, n_u, cbody, 0)

    return scatter_k(tp, dst, zeros_nd)


def _combine_body(p_ref, s_ref, o_ref):
    o_ref[...] = (p_ref[0] + p_ref[1]) / s_ref[0, 0]


def _combine(partials, sq):
    N = partials.shape[1]
    BN = 1000
    return pl.pallas_call(
        _combine_body,
        grid=(N // BN,),
        in_specs=[
            pl.BlockSpec((2, BN, DIM), lambda i: (0, i, 0)),
            pl.BlockSpec(memory_space=pltpu.SMEM),
        ],
        out_specs=pl.BlockSpec((BN, DIM), lambda i: (i, 0)),
        out_shape=jax.ShapeDtypeStruct((N, DIM), jnp.float32),
    )(partials, sq)


def kernel(node_features, edge_sh_features, edge_radial_features, edge_src,
           edge_dst, n_nodes, W1, W2):
    N = node_features.shape[0]
    RAD = edge_radial_features.shape[1]
    HID = W1.shape[1]

    rp = jnp.pad(edge_radial_features, ((0, 0), (0, 16 - RAD)))
    W1p = jnp.pad(W1 * (SQRT2 / math.sqrt(float(RAD))),
                  ((0, 16 - RAD), (0, 0))).astype(jnp.bfloat16)
    W2b = ((W2 * (1.0 / math.sqrt(float(HID))))[:, _PERM_NP]
           ).astype(jnp.bfloat16)
    bf = jnp.bfloat16
    consts = (jnp.asarray(_S_NP, bf),
              jnp.asarray(_Q_NP, bf), jnp.asarray(_P_NP, bf))

    x_e = _sc_gather(node_features, edge_src)
    tp = _dense_call(rp, x_e, edge_sh_features, W1p, W2b, consts)
    partials = _sc_scatter(tp, edge_dst, jnp.zeros((N, DIM), jnp.float32))
    sq = jnp.sqrt((jnp.asarray(n_nodes) - 1).astype(jnp.float32)).reshape(1, 1)
    return _combine(partials, sq)


# speedup vs baseline: 2.7708x; 1.0184x over previous
import functools
import math

import numpy as np
import jax
import jax.numpy as jnp
from jax import lax
from jax.experimental import pallas as pl
from jax.experimental.pallas import tpu as pltpu
from jax.experimental.pallas import tpu_sc as plsc

MUL = 32
DIM = 4 * MUL
SQRT2 = math.sqrt(2.0)
INV_SQRT3 = 1.0 / math.sqrt(3.0)
ALPHA = 1.0 / math.sqrt(2.0 * MUL)

TE = 1280
CE = 128


def _build_consts():
    c = np.arange(32 * MUL)
    Sm = (c[:, None] // MUL == np.arange(MUL)[None, :]).astype(np.float32)
    Qm = np.zeros((3 * MUL, 3 * MUL), np.float32)
    for i in range(MUL):
        for k in range(3):
            Qm[3 * i + k, k * MUL + i] = 1.0
    Pm = np.zeros((3 * MUL, 3 * MUL), np.float32)
    for j in range(MUL):
        for k in range(3):
            Pm[k * MUL + j, 3 * j + k] = 1.0
    perm = np.concatenate(
        [p * MUL * MUL + (c % MUL) * MUL + c // MUL for p in range(4)])
    return Sm, Qm, Pm, perm

_S_NP, _Q_NP, _P_NP, _PERM_NP = _build_consts()


def _dense_body(r_ref, x_ref, sh_ref, w1_ref, w2_ref, S_ref, Q_ref,
                P_ref, o_ref):
    f32 = jnp.float32
    bf16 = jnp.bfloat16
    rb = r_ref[...].astype(bf16)
    hb = jnp.maximum(
        jnp.dot(rb, w1_ref[...], preferred_element_type=f32),
        0.0).astype(bf16)

    x = x_ref[...]
    sh = sh_ref[...]
    s = x[:, :MUL]
    xv = x[:, MUL:].astype(bf16)
    y0 = sh[:, 0:1]

    Sm = S_ref[...]
    v_all = jnp.dot(xv, Q_ref[...], preferred_element_type=f32)
    v0, v1, v2 = v_all[:, :MUL], v_all[:, MUL:2 * MUL], v_all[:, 2 * MUL:]
    dv = (v0 * sh[:, 1:2] + v1 * sh[:, 2:3] + v2 * sh[:, 3:4]) * INV_SQRT3

    def rep(z):
        return pltpu.repeat(z, MUL, axis=1)

    w2 = w2_ref[...]
    B = MUL * MUL

    def wblk(p):
        return jnp.dot(hb, w2[:, p * B:(p + 1) * B], preferred_element_type=f32)

    def contract(zrep, wp):
        return jnp.dot((zrep * wp).astype(bf16), Sm, preferred_element_type=f32)

    srep = rep(s)
    q1 = contract(srep, wblk(0))
    q2 = contract(rep(dv), wblk(1))
    q3 = contract(srep, wblk(2))
    w4 = wblk(3)
    q40 = contract(rep(v0), w4)
    q41 = contract(rep(v1), w4)
    q42 = contract(rep(v2), w4)

    out_s = ALPHA * (q1 * y0 + q2)
    val = jnp.concatenate([
        ALPHA * (q3 * sh[:, 1:2] + q40 * y0),
        ALPHA * (q3 * sh[:, 2:3] + q41 * y0),
        ALPHA * (q3 * sh[:, 3:4] + q42 * y0),
    ], axis=1).astype(bf16)
    out_vec = jnp.dot(val, P_ref[...], preferred_element_type=f32)
    o_ref[:, :MUL] = out_s
    o_ref[:, MUL:] = out_vec


def _dense_call(rp, x_e, sh, W1p, W2b, consts):
    E = rp.shape[0]
    HID = W1p.shape[1]
    Sm, Qm, Pm = consts
    grid = (E // TE,)
    return pl.pallas_call(
        _dense_body,
        grid=grid,
        in_specs=[
            pl.BlockSpec((TE, 16), lambda i: (i, 0)),
            pl.BlockSpec((TE, DIM), lambda i: (i, 0)),
            pl.BlockSpec((TE, 4), lambda i: (i, 0)),
            pl.BlockSpec((16, HID), lambda i: (0, 0)),
            pl.BlockSpec((HID, 4 * MUL * MUL), lambda i: (0, 0)),
            pl.BlockSpec((MUL * MUL, MUL), lambda i: (0, 0)),
            pl.BlockSpec((3 * MUL, 3 * MUL), lambda i: (0, 0)),
            pl.BlockSpec((3 * MUL, 3 * MUL), lambda i: (0, 0)),
        ],
        out_specs=pl.BlockSpec((TE, DIM), lambda i: (i, 0)),
        out_shape=jax.ShapeDtypeStruct((E, DIM), jnp.float32),
    )(rp, x_e, sh, W1p, W2b, Sm, Qm, Pm)


def _sc_gather(table, idx):
    E = idx.shape[0]
    n_chunks = E // CE
    mesh = plsc.VectorSubcoreMesh(core_axis_name="c", subcore_axis_name="s")
    NW = 32
    base_t, extra = divmod(n_chunks, NW)

    @functools.partial(
        pl.kernel,
        out_type=jax.ShapeDtypeStruct((E, DIM), jnp.float32),
        mesh=mesh,
        scratch_types=[
            pltpu.VMEM((CE,), jnp.int32),
            pltpu.VMEM((CE, DIM), jnp.float32),
            pltpu.SemaphoreType.DMA,
        ],
    )
    def gather_k(table_hbm, idx_hbm, out_hbm, idx_v, rows_v, sem):
        wid = lax.axis_index("s") * 2 + lax.axis_index("c")
        n_t = base_t + jnp.where(wid < extra, 1, 0)

        def body(t, carry):
            off = (wid + NW * t) * CE
            pltpu.sync_copy(idx_hbm.at[pl.ds(off, CE)], idx_v)
            pltpu.async_copy(table_hbm.at[idx_v], rows_v, sem).wait()
            pltpu.sync_copy(rows_v, out_hbm.at[pl.ds(off, CE)])
            return carry

        lax.fori_loop(0, n_t, body, 0)

    return gather_k(table, idx)


def _sc_scatter(tp, dst, zeros_nd):
    E = tp.shape[0]
    N = zeros_nd.shape[0]
    n_chunks = E // CE
    mesh = plsc.VectorSubcoreMesh(core_axis_name="c", subcore_axis_name="s")
    NW = 32
    NS = 16
    base_t, extra = divmod(n_chunks, NW)
    CR = 16
    base_u, extra_u = divmod(N // CR, NS)

    @functools.partial(
        pl.kernel,
        out_type=jax.ShapeDtypeStruct((2, N, DIM), jnp.float32),
        mesh=mesh,
        scratch_types=[
            pltpu.VMEM((CE,), jnp.int32),
            pltpu.VMEM((CE, DIM), jnp.float32),
            pltpu.VMEM_SHARED((N, DIM), jnp.float32),
            pltpu.SemaphoreType.DMA,
        ],
    )
    def scatter_k(tp_hbm, dst_hbm, zeros_hbm, out_hbm, idx_v, rows_v, acc_sh,
                  sem):
        cid = lax.axis_index("c")
        sid = lax.axis_index("s")
        wid = sid * 2 + cid

        @pl.when(sid == 0)
        def _():
            pltpu.sync_copy(zeros_hbm, acc_sh)

        plsc.subcore_barrier()

        n_t = base_t + jnp.where(wid < extra, 1, 0)

        def body(t, carry):
            off = (wid + NW * t) * CE
            pltpu.sync_copy(dst_hbm.at[pl.ds(off, CE)], idx_v)
            pltpu.sync_copy(tp_hbm.at[pl.ds(off, CE)], rows_v)
            pltpu.sync_copy(rows_v, acc_sh.at[idx_v], add=True)
            return carry

        lax.fori_loop(0, n_t, body, 0)
        plsc.subcore_barrier()

        n_u = base_u + jnp.where(sid < extra_u, 1, 0)

        def cbody(u, carry):
            roff = (sid + NS * u) * CR
            pltpu.sync_copy(acc_sh.at[pl.ds(roff, CR)],
                            out_hbm.at[cid, pl.ds(roff, CR)])
            return carry

        lax.fori_loop(0, n_u, cbody, 0)

    return scatter_k(tp, dst, zeros_nd)


def _combine_body(p_ref, s_ref, o_ref):
    o_ref[...] = (p_ref[0] + p_ref[1]) / s_ref[0, 0]


def _combine(partials, sq):
    N = partials.shape[1]
    BN = 1000
    return pl.pallas_call(
        _combine_body,
        grid=(N // BN,),
        in_specs=[
            pl.BlockSpec((2, BN, DIM), lambda i: (0, i, 0)),
            pl.BlockSpec(memory_space=pltpu.SMEM),
        ],
        out_specs=pl.BlockSpec((BN, DIM), lambda i: (i, 0)),
        out_shape=jax.ShapeDtypeStruct((N, DIM), jnp.float32),
    )(partials, sq)


def kernel(node_features, edge_sh_features, edge_radial_features, edge_src,
           edge_dst, n_nodes, W1, W2):
    N = node_features.shape[0]
    RAD = edge_radial_features.shape[1]
    HID = W1.shape[1]

    rp = jnp.pad(edge_radial_features, ((0, 0), (0, 16 - RAD)))
    W1p = jnp.pad(W1 * (SQRT2 / math.sqrt(float(RAD))),
                  ((0, 16 - RAD), (0, 0))).astype(jnp.bfloat16)
    W2b = ((W2 * (1.0 / math.sqrt(float(HID))))[:, _PERM_NP]
           ).astype(jnp.bfloat16)
    bf = jnp.bfloat16
    consts = (jnp.asarray(_S_NP, bf),
              jnp.asarray(_Q_NP, bf), jnp.asarray(_P_NP, bf))

    x_e = _sc_gather(node_features, edge_src)
    tp = _dense_call(rp, x_e, edge_sh_features, W1p, W2b, consts)
    partials = _sc_scatter(tp, edge_dst, jnp.zeros((N, DIM), jnp.float32))
    sq = jnp.sqrt((jnp.asarray(n_nodes) - 1).astype(jnp.float32)).reshape(1, 1)
    return _combine(partials, sq)


# speedup vs baseline: 2.8199x; 1.0177x over previous
import functools
import math

import numpy as np
import jax
import jax.numpy as jnp
from jax import lax
from jax.experimental import pallas as pl
from jax.experimental.pallas import tpu as pltpu
from jax.experimental.pallas import tpu_sc as plsc

MUL = 32
DIM = 4 * MUL
SQRT2 = math.sqrt(2.0)
INV_SQRT3 = 1.0 / math.sqrt(3.0)
ALPHA = 1.0 / math.sqrt(2.0 * MUL)

TE = 1280
CE = 128


def _build_consts():
    c = np.arange(32 * MUL)
    Sm = (c[:, None] // MUL == np.arange(MUL)[None, :]).astype(np.float32)
    Qm = np.zeros((3 * MUL, 3 * MUL), np.float32)
    for i in range(MUL):
        for k in range(3):
            Qm[3 * i + k, k * MUL + i] = 1.0
    Pm = np.zeros((3 * MUL, 3 * MUL), np.float32)
    for j in range(MUL):
        for k in range(3):
            Pm[k * MUL + j, 3 * j + k] = 1.0
    perm = np.concatenate(
        [p * MUL * MUL + (c % MUL) * MUL + c // MUL for p in range(4)])
    return Sm, Qm, Pm, perm

_S_NP, _Q_NP, _P_NP, _PERM_NP = _build_consts()


def _dense_body(r_ref, x_ref, sh_ref, w1_ref, w2_ref, S_ref, Q_ref,
                P_ref, o_ref):
    f32 = jnp.float32
    bf16 = jnp.bfloat16
    rb = r_ref[...].astype(bf16)
    hb = jnp.maximum(
        jnp.dot(rb, w1_ref[...], preferred_element_type=f32),
        0.0).astype(bf16)

    x = x_ref[...]
    sh = sh_ref[...]
    s = x[:, :MUL]
    xv = x[:, MUL:].astype(bf16)
    y0 = sh[:, 0:1]

    Sm = S_ref[...]
    v_all = jnp.dot(xv, Q_ref[...], preferred_element_type=f32)
    v0, v1, v2 = v_all[:, :MUL], v_all[:, MUL:2 * MUL], v_all[:, 2 * MUL:]
    dv = (v0 * sh[:, 1:2] + v1 * sh[:, 2:3] + v2 * sh[:, 3:4]) * INV_SQRT3

    def rep(z):
        return pltpu.repeat(z, MUL, axis=1)

    w2 = w2_ref[...]
    B = MUL * MUL

    def wblk(p):
        return jnp.dot(hb, w2[:, p * B:(p + 1) * B], preferred_element_type=f32)

    def contract(zrep, wp):
        return jnp.dot((zrep * wp).astype(bf16), Sm, preferred_element_type=f32)

    srep = rep(s)
    q1 = contract(srep, wblk(0))
    q2 = contract(rep(dv), wblk(1))
    q3 = contract(srep, wblk(2))
    w4 = wblk(3)
    q40 = contract(rep(v0), w4)
    q41 = contract(rep(v1), w4)
    q42 = contract(rep(v2), w4)

    out_s = ALPHA * (q1 * y0 + q2)
    val = jnp.concatenate([
        ALPHA * (q3 * sh[:, 1:2] + q40 * y0),
        ALPHA * (q3 * sh[:, 2:3] + q41 * y0),
        ALPHA * (q3 * sh[:, 3:4] + q42 * y0),
    ], axis=1).astype(bf16)
    out_vec = jnp.dot(val, P_ref[...], preferred_element_type=f32)
    o_ref[:, :MUL] = out_s
    o_ref[:, MUL:] = out_vec


def _dense_call(rp, x_e, sh, W1p, W2b, consts):
    E = rp.shape[0]
    HID = W1p.shape[1]
    Sm, Qm, Pm = consts
    grid = (E // TE,)
    return pl.pallas_call(
        _dense_body,
        grid=grid,
        in_specs=[
            pl.BlockSpec((TE, 16), lambda i: (i, 0)),
            pl.BlockSpec((TE, DIM), lambda i: (i, 0)),
            pl.BlockSpec((TE, 4), lambda i: (i, 0)),
            pl.BlockSpec((16, HID), lambda i: (0, 0)),
            pl.BlockSpec((HID, 4 * MUL * MUL), lambda i: (0, 0)),
            pl.BlockSpec((MUL * MUL, MUL), lambda i: (0, 0)),
            pl.BlockSpec((3 * MUL, 3 * MUL), lambda i: (0, 0)),
            pl.BlockSpec((3 * MUL, 3 * MUL), lambda i: (0, 0)),
        ],
        out_specs=pl.BlockSpec((TE, DIM), lambda i: (i, 0)),
        out_shape=jax.ShapeDtypeStruct((E, DIM), jnp.float32),
    )(rp, x_e, sh, W1p, W2b, Sm, Qm, Pm)


def _sc_gather(table, idx):
    E = idx.shape[0]
    n_chunks = E // CE
    mesh = plsc.VectorSubcoreMesh(core_axis_name="c", subcore_axis_name="s")
    NW = 32
    base_t, extra = divmod(n_chunks, NW)

    @functools.partial(
        pl.kernel,
        out_type=jax.ShapeDtypeStruct((E, DIM), jnp.float32),
        mesh=mesh,
        scratch_types=[
            pltpu.VMEM((2, CE), jnp.int32),
            pltpu.VMEM((2, CE, DIM), jnp.float32),
            pltpu.SemaphoreType.DMA,
            pltpu.SemaphoreType.DMA,
            pltpu.SemaphoreType.DMA,
        ],
    )
    def gather_k(table_hbm, idx_hbm, out_hbm, idx_v, rows_v, semI, semG, semO):
        wid = lax.axis_index("s") * 2 + lax.axis_index("c")
        n_t = base_t + jnp.where(wid < extra, 1, 0)

        def off(t):
            return (wid + NW * t) * CE

        pltpu.async_copy(idx_hbm.at[pl.ds(off(0), CE)], idx_v.at[0], semI)

        def body(t, carry):
            b = lax.rem(t, 2)
            pltpu.make_async_copy(idx_hbm.at[pl.ds(off(t), CE)],
                                  idx_v.at[b], semI).wait()

            @pl.when(t + 1 < n_t)
            def _():
                pltpu.async_copy(idx_hbm.at[pl.ds(off(t + 1), CE)],
                                 idx_v.at[1 - b], semI)

            @pl.when(t >= 2)
            def _():
                pltpu.make_async_copy(rows_v.at[b],
                                      out_hbm.at[pl.ds(off(t - 2), CE)],
                                      semO).wait()

            pltpu.async_copy(table_hbm.at[idx_v.at[b]], rows_v.at[b],
                             semG).wait()
            pltpu.async_copy(rows_v.at[b], out_hbm.at[pl.ds(off(t), CE)], semO)
            return carry

        lax.fori_loop(0, n_t, body, 0)
        b2 = lax.rem(n_t, 2)
        pltpu.make_async_copy(rows_v.at[b2],
                              out_hbm.at[pl.ds(off(n_t - 2), CE)], semO).wait()
        pltpu.make_async_copy(rows_v.at[1 - b2],
                              out_hbm.at[pl.ds(off(n_t - 1), CE)], semO).wait()

    return gather_k(table, idx)


def _sc_scatter(tp, dst, zeros_nd):
    E = tp.shape[0]
    N = zeros_nd.shape[0]
    n_chunks = E // CE
    mesh = plsc.VectorSubcoreMesh(core_axis_name="c", subcore_axis_name="s")
    NW = 32
    NS = 16
    base_t, extra = divmod(n_chunks, NW)
    CR = 16
    base_u, extra_u = divmod(N // CR, NS)

    @functools.partial(
        pl.kernel,
        out_type=jax.ShapeDtypeStruct((2, N, DIM), jnp.float32),
        mesh=mesh,
        scratch_types=[
            pltpu.VMEM((2, CE), jnp.int32),
            pltpu.VMEM((2, CE, DIM), jnp.float32),
            pltpu.VMEM_SHARED((N, DIM), jnp.float32),
            pltpu.SemaphoreType.DMA,
            pltpu.SemaphoreType.DMA,
            pltpu.SemaphoreType.DMA,
        ],
    )
    def scatter_k(tp_hbm, dst_hbm, zeros_hbm, out_hbm, idx_v, rows_v, acc_sh,
                  semI, semT, semA):
        cid = lax.axis_index("c")
        sid = lax.axis_index("s")
        wid = sid * 2 + cid

        @pl.when(sid == 0)
        def _():
            pltpu.sync_copy(zeros_hbm, acc_sh)

        plsc.subcore_barrier()

        n_t = base_t + jnp.where(wid < extra, 1, 0)

        def off(t):
            return (wid + NW * t) * CE

        pltpu.async_copy(dst_hbm.at[pl.ds(off(0), CE)], idx_v.at[0], semI)
        pltpu.async_copy(tp_hbm.at[pl.ds(off(0), CE)], rows_v.at[0], semT)

        def body(t, carry):
            b = lax.rem(t, 2)
            pltpu.make_async_copy(dst_hbm.at[pl.ds(off(t), CE)],
                                  idx_v.at[b], semI).wait()
            pltpu.make_async_copy(tp_hbm.at[pl.ds(off(t), CE)],
                                  rows_v.at[b], semT).wait()

            @pl.when(t + 1 < n_t)
            def _():
                pltpu.async_copy(dst_hbm.at[pl.ds(off(t + 1), CE)],
                                 idx_v.at[1 - b], semI)
                pltpu.async_copy(tp_hbm.at[pl.ds(off(t + 1), CE)],
                                 rows_v.at[1 - b], semT)

            pltpu.async_copy(rows_v.at[b], acc_sh.at[idx_v.at[b]], semA,
                             add=True).wait()
            return carry

        lax.fori_loop(0, n_t, body, 0)
        plsc.subcore_barrier()

        n_u = base_u + jnp.where(sid < extra_u, 1, 0)

        def cbody(u, carry):
            roff = (sid + NS * u) * CR
            pltpu.async_copy(acc_sh.at[pl.ds(roff, CR)],
                             out_hbm.at[cid, pl.ds(roff, CR)], semT)
            return carry

        lax.fori_loop(0, n_u, cbody, 0)

        def dbody(u, carry):
            roff = (sid + NS * u) * CR
            pltpu.make_async_copy(acc_sh.at[pl.ds(roff, CR)],
                                  out_hbm.at[cid, pl.ds(roff, CR)],
                                  semT).wait()
            return carry

        lax.fori_loop(0, n_u, dbody, 0)

    return scatter_k(tp, dst, zeros_nd)


def _combine_body(p_ref, s_ref, o_ref):
    o_ref[...] = (p_ref[0] + p_ref[1]) / s_ref[0, 0]


def _combine(partials, sq):
    N = partials.shape[1]
    BN = 1000
    return pl.pallas_call(
        _combine_body,
        grid=(N // BN,),
        in_specs=[
            pl.BlockSpec((2, BN, DIM), lambda i: (0, i, 0)),
            pl.BlockSpec(memory_space=pltpu.SMEM),
        ],
        out_specs=pl.BlockSpec((BN, DIM), lambda i: (i, 0)),
        out_shape=jax.ShapeDtypeStruct((N, DIM), jnp.float32),
    )(partials, sq)


def kernel(node_features, edge_sh_features, edge_radial_features, edge_src,
           edge_dst, n_nodes, W1, W2):
    N = node_features.shape[0]
    RAD = edge_radial_features.shape[1]
    HID = W1.shape[1]

    rp = jnp.pad(edge_radial_features, ((0, 0), (0, 16 - RAD)))
    W1p = jnp.pad(W1 * (SQRT2 / math.sqrt(float(RAD))),
                  ((0, 16 - RAD), (0, 0))).astype(jnp.bfloat16)
    W2b = ((W2 * (1.0 / math.sqrt(float(HID))))[:, _PERM_NP]
           ).astype(jnp.bfloat16)
    bf = jnp.bfloat16
    consts = (jnp.asarray(_S_NP, bf),
              jnp.asarray(_Q_NP, bf), jnp.asarray(_P_NP, bf))

    x_e = _sc_gather(node_features, edge_src)
    tp = _dense_call(rp, x_e, edge_sh_features, W1p, W2b, consts)
    partials = _sc_scatter(tp, edge_dst, jnp.zeros((N, DIM), jnp.float32))
    sq = jnp.sqrt((jnp.asarray(n_nodes) - 1).astype(jnp.float32)).reshape(1, 1)
    return _combine(partials, sq)
